# baseline (speedup 1.0000x reference)
import functools

import jax
import jax.numpy as jnp
from jax import lax
from jax.experimental import pallas as pl
from jax.experimental.pallas import tpu as pltpu
from jax.experimental.pallas import tpu_sc as plsc

CUTOFF = 10.0
CUTOFF_SR = 2.0
KEHALF = 7.199822675975274

NC = 2
NS = 16
L = 16
NW = NC * NS

CHUNK = 4000
PIECE = 1600


def _rsqrt(x):
  i = lax.bitcast_convert_type(x, jnp.int32)
  i = jnp.int32(0x5F3759DF) - lax.shift_right_logical(i, 1)
  y = lax.bitcast_convert_type(i, jnp.float32)
  for _ in range(2):
    y = y * (1.5 - 0.5 * x * y * y)
  return y


@functools.lru_cache(maxsize=None)
def _build(n_nodes, n_edges, chunk):
  n_work = n_edges // NW
  n_chunks = n_work // chunk
  stage = n_nodes // NS
  groups = chunk // L

  mesh = plsc.VectorSubcoreMesh(core_axis_name="c", subcore_axis_name="s")

  def body(wa_hbm, wb_hbm, iu_hbm, iv_hbm,
           qx_hbm, yz_hbm,
           out_hbm,
           qx_sh, yz_sh,
           i20, wa0, wb0,
           g10, g20, o0,
           si0, sn0, sg0, so0,
           i21, wa1, wb1,
           g11, g21, o1,
           si1, sn1, sg1, so1):
    cid = lax.axis_index("c")
    sid = lax.axis_index("s")
    wid = cid * NS + sid

    sets = [
        dict(i2=i20, wa=wa0, wb=wb0,
             g1=g10, g2=g20, out=o0,
             si=si0, sn=sn0, sg=sg0, so=so0),
        dict(i2=i21, wa=wa1, wb=wb1,
             g1=g11, g2=g21, out=o1,
             si=si1, sn=sn1, sg=sg1, so=so1),
    ]

    n_piece = stage // PIECE
    for hbm, sh, bounce in ((qx_hbm, qx_sh, g10), (yz_hbm, yz_sh, g20)):
      for p in range(n_piece):
        off = sid * stage + p * PIECE
        pltpu.sync_copy(hbm.at[pl.ds(off, PIECE)], bounce.at[pl.ds(0, PIECE)])
        pltpu.sync_copy(bounce.at[pl.ds(0, PIECE)], sh.at[pl.ds(off, PIECE)])
    plsc.subcore_barrier()

    def esl(j):
      return pl.ds(wid * n_work + j * chunk, chunk)

    drain_sl = pl.ds(0, chunk)

    def fire_lin(s, j):
      sl = esl(j)
      pltpu.async_copy(iu_hbm.at[sl], s["i2"].at[pl.ds(0, chunk)], s["si"])
      pltpu.async_copy(iv_hbm.at[sl], s["i2"].at[pl.ds(chunk, chunk)], s["si"])
      pltpu.async_copy(wa_hbm.at[sl], s["wa"], s["sn"])
      pltpu.async_copy(wb_hbm.at[sl], s["wb"], s["sn"])

    def drain_idx(s):
      pltpu.make_async_copy(iu_hbm.at[pl.ds(0, 2 * chunk)], s["i2"],
                            s["si"]).wait()

    def drain_in(s):
      for r in ("wa", "wb"):
        pltpu.make_async_copy(wa_hbm.at[drain_sl], s[r], s["sn"]).wait()

    def fire_gath(s):
      iu_ix = s["i2"].at[pl.ds(0, chunk)]
      iv_ix = s["i2"].at[pl.ds(chunk, chunk)]
      pltpu.async_copy(qx_sh.at[iu_ix], s["g1"].at[pl.ds(0, chunk)], s["sg"])
      pltpu.async_copy(qx_sh.at[iv_ix], s["g1"].at[pl.ds(chunk, chunk)],
                       s["sg"])
      pltpu.async_copy(yz_sh.at[iu_ix], s["g2"].at[pl.ds(0, chunk)], s["sg"])
      pltpu.async_copy(yz_sh.at[iv_ix], s["g2"].at[pl.ds(chunk, chunk)],
                       s["sg"])

    def drain_gath(s):
      for r in ("g1", "g2"):
        pltpu.make_async_copy(iu_hbm.at[pl.ds(0, 2 * chunk)], s[r],
                              s["sg"]).wait()

    def fire_out(s, j):
      pltpu.async_copy(s["out"], out_hbm.at[esl(j)], s["so"])

    def drain_out(s):
      pltpu.make_async_copy(wa_hbm.at[drain_sl], s["out"], s["so"]).wait()

    def comp(s):
      wa_v, wb_v = s["wa"], s["wb"]
      g1_v, g2_v = s["g1"], s["g2"]
      out_v = s["out"]

      hi_mask = jnp.full((L,), -65536, jnp.int32)


      @plsc.parallel_loop(0, groups, 1, unroll=2)
      def compute(i):
        base = i * L
        wa = wa_v[pl.ds(base, L)]
        wb = wb_v[pl.ds(base, L)]
        d = lax.bitcast_convert_type(wa & hi_mask, jnp.float32)
        vx = lax.bitcast_convert_type(lax.shift_left(wa, 16), jnp.float32)
        vy = lax.bitcast_convert_type(wb & hi_mask, jnp.float32)
        vz = lax.bitcast_convert_type(lax.shift_left(wb, 16), jnp.float32)
        w1u = g1_v[pl.ds(base, L)]
        w2u = g2_v[pl.ds(base, L)]
        qu = lax.bitcast_convert_type(w1u & hi_mask, jnp.float32)
        mux = lax.bitcast_convert_type(lax.shift_left(w1u, 16), jnp.float32)
        muy = lax.bitcast_convert_type(w2u & hi_mask, jnp.float32)
        muz = lax.bitcast_convert_type(lax.shift_left(w2u, 16), jnp.float32)
        w1v = g1_v[pl.ds(chunk + base, L)]
        w2v = g2_v[pl.ds(chunk + base, L)]
        qv = lax.bitcast_convert_type(w1v & hi_mask, jnp.float32)
        mvx = lax.bitcast_convert_type(lax.shift_left(w1v, 16), jnp.float32)
        mvy = lax.bitcast_convert_type(w2v & hi_mask, jnp.float32)
        mvz = lax.bitcast_convert_type(lax.shift_left(w2v, 16), jnp.float32)

        x = jnp.clip(d * (1.0 / CUTOFF_SR), 0.0, 1.0)
        x2 = x * x
        x3 = x2 * x
        sw = 1.0 - (6.0 * x2 - 15.0 * x + 10.0) * x3
        inv_d = 1.0 / d
        chi = sw * _rsqrt(d * d + 1.0) + (1.0 - sw) * inv_d
        chi2 = chi * chi
        chi3 = chi2 * chi

        s1 = 1.0 / CUTOFF
        s2 = s1 * s1
        s3 = s2 * s1

        dot_uv = (vx * mvx + vy * mvy + vz * mvz) * inv_d
        dot_vu = (vx * mux + vy * muy + vz * muz) * inv_d
        mumu = mux * mvx + muy * mvy + muz * mvz

        e = qu * qv * (chi - s1)
        e = e + 2.0 * qu * dot_uv * (chi2 - s2)
        e = e + (mumu - 3.0 * dot_uv * dot_vu) * (chi3 - s3)
        e = KEHALF * e
        e = jnp.where(d <= CUTOFF, e, jnp.zeros_like(e))
        out_v[pl.ds(base, L)] = e

    fire_lin(sets[0], 0)
    drain_idx(sets[0])
    fire_gath(sets[0])
    fire_lin(sets[1], 1)

    def pair(jj, _):
      for p in (0, 1):
        s = sets[p]
        t = sets[1 - p]
        j = 2 * jj + p

        @pl.when(j + 1 < n_chunks)
        def _():
          drain_idx(t)
          fire_gath(t)

        drain_in(s)
        drain_gath(s)

        @pl.when(j >= 2)
        def _():
          drain_out(s)

        comp(s)
        fire_out(s, j)

        @pl.when(j + 2 < n_chunks)
        def _():
          fire_lin(s, j + 2)
      return ()

    lax.fori_loop(0, n_chunks // 2, pair, (), unroll=False)
    drain_out(sets[0])
    drain_out(sets[1])

  vm_f = pltpu.VMEM((chunk,), jnp.float32)
  vm_i = pltpu.VMEM((chunk,), jnp.int32)
  vm_i2 = pltpu.VMEM((2 * chunk,), jnp.int32)
  sem = pltpu.SemaphoreType.DMA
  one_set = ([vm_i2] + [vm_i] * 2 + [vm_i2, vm_i2] + [vm_f]
             + [sem] * 4)

  return pl.kernel(
      body,
      out_type=jax.ShapeDtypeStruct((n_edges,), jnp.float32),
      mesh=mesh,
      scratch_types=(
          [pltpu.VMEM_SHARED((n_nodes,), jnp.int32),
           pltpu.VMEM_SHARED((n_nodes,), jnp.int32)]
          + one_set + one_set
      ),
  )


def kernel(distances_uv, atomic_charges, idx_u, idx_v, vectors_uv,
           atomic_dipoles):
  n_edges = distances_uv.shape[0]
  n_nodes = atomic_charges.shape[0]

  n_pad = (-n_nodes) % (NS * PIECE)
  q = atomic_charges
  dip = atomic_dipoles
  if n_pad:
    q = jnp.pad(q, (0, n_pad))
    dip = jnp.pad(dip, ((0, n_pad), (0, 0)))
  def b16(a):
    return lax.bitcast_convert_type(a.astype(jnp.bfloat16),
                                    jnp.uint16).astype(jnp.uint32)

  qx = ((b16(q) << 16) | b16(dip[:, 0])).astype(jnp.int32)
  yz = ((b16(dip[:, 1]) << 16) | b16(dip[:, 2])).astype(jnp.int32)

  iu = idx_u.astype(jnp.int32)
  iv = idx_v.astype(jnp.int32)
  d = distances_uv
  vec = vectors_uv

  e_pad = (-n_edges) % (NW * CHUNK * 2)
  if e_pad:
    d = jnp.pad(d, (0, e_pad), constant_values=1.0)
    vec = jnp.pad(vec, ((0, e_pad), (0, 0)))
    iu = jnp.pad(iu, (0, e_pad))
    iv = jnp.pad(iv, (0, e_pad))

  vec_t = vec.T
  wa = ((b16(d) << 16) | b16(vec_t[0])).astype(jnp.int32)
  wb = ((b16(vec_t[1]) << 16) | b16(vec_t[2])).astype(jnp.int32)

  fn = _build(q.shape[0], d.shape[0], CHUNK)
  out = fn(wa, wb, iu, iv, qx, yz)
  return out[:n_edges] if e_pad else out


# speedup vs baseline: 1.0878x; 1.0878x over previous
import functools

import jax
import jax.numpy as jnp
from jax import lax
from jax.experimental import pallas as pl
from jax.experimental.pallas import tpu as pltpu
from jax.experimental.pallas import tpu_sc as plsc

CUTOFF = 10.0
CUTOFF_SR = 2.0
KEHALF = 7.199822675975274

NC = 2
NS = 16
L = 16
NW = NC * NS

CHUNK = 2000
PIECE = 1600


def _rsqrt(x):
  i = lax.bitcast_convert_type(x, jnp.int32)
  i = jnp.int32(0x5F3759DF) - lax.shift_right_logical(i, 1)
  y = lax.bitcast_convert_type(i, jnp.float32)
  for _ in range(2):
    y = y * (1.5 - 0.5 * x * y * y)
  return y


@functools.lru_cache(maxsize=None)
def _build(n_nodes, n_edges, chunk):
  n_work = n_edges // NW
  n_chunks = n_work // chunk
  stage = n_nodes // NS
  groups = chunk // L

  mesh = plsc.VectorSubcoreMesh(core_axis_name="c", subcore_axis_name="s")

  def body(d_hbm, vx_hbm, vy_hbm, vz_hbm, iu_hbm, iv_hbm,
           qx_hbm, yz_hbm,
           out_hbm,
           qx_sh, yz_sh,
           i20, d0, vx0, vy0, vz0,
           g10, g20, o0,
           si0, sn0, sg0, so0,
           i21, d1, vx1, vy1, vz1,
           g11, g21, o1,
           si1, sn1, sg1, so1):
    cid = lax.axis_index("c")
    sid = lax.axis_index("s")
    wid = cid * NS + sid

    sets = [
        dict(i2=i20, d=d0, vx=vx0, vy=vy0, vz=vz0,
             g1=g10, g2=g20, out=o0,
             si=si0, sn=sn0, sg=sg0, so=so0),
        dict(i2=i21, d=d1, vx=vx1, vy=vy1, vz=vz1,
             g1=g11, g2=g21, out=o1,
             si=si1, sn=sn1, sg=sg1, so=so1),
    ]

    n_piece = stage // PIECE
    for hbm, sh, bounce in ((qx_hbm, qx_sh, g10), (yz_hbm, yz_sh, g20)):
      for p in range(n_piece):
        off = sid * stage + p * PIECE
        pltpu.sync_copy(hbm.at[pl.ds(off, PIECE)], bounce.at[pl.ds(0, PIECE)])
        pltpu.sync_copy(bounce.at[pl.ds(0, PIECE)], sh.at[pl.ds(off, PIECE)])
    plsc.subcore_barrier()

    def esl(j):
      return pl.ds(wid * n_work + j * chunk, chunk)

    drain_sl = pl.ds(0, chunk)

    def fire_lin(s, j):
      sl = esl(j)
      pltpu.async_copy(iu_hbm.at[sl], s["i2"].at[pl.ds(0, chunk)], s["si"])
      pltpu.async_copy(iv_hbm.at[sl], s["i2"].at[pl.ds(chunk, chunk)], s["si"])
      pltpu.async_copy(d_hbm.at[sl], s["d"], s["sn"])
      pltpu.async_copy(vx_hbm.at[sl], s["vx"], s["sn"])
      pltpu.async_copy(vy_hbm.at[sl], s["vy"], s["sn"])
      pltpu.async_copy(vz_hbm.at[sl], s["vz"], s["sn"])

    def drain_idx(s):
      pltpu.make_async_copy(iu_hbm.at[pl.ds(0, 2 * chunk)], s["i2"],
                            s["si"]).wait()

    def drain_in(s):
      for r in ("d", "vx", "vy", "vz"):
        pltpu.make_async_copy(d_hbm.at[drain_sl], s[r], s["sn"]).wait()

    def fire_gath(s):
      pltpu.async_copy(qx_sh.at[s["i2"]], s["g1"], s["sg"])
      pltpu.async_copy(yz_sh.at[s["i2"]], s["g2"], s["sg"])

    def drain_gath(s):
      for r in ("g1", "g2"):
        pltpu.make_async_copy(iu_hbm.at[pl.ds(0, 2 * chunk)], s[r],
                              s["sg"]).wait()

    def fire_out(s, j):
      pltpu.async_copy(s["out"], out_hbm.at[esl(j)], s["so"])

    def drain_out(s):
      pltpu.make_async_copy(d_hbm.at[drain_sl], s["out"], s["so"]).wait()

    def comp(s):
      d_v, vx_v, vy_v, vz_v = s["d"], s["vx"], s["vy"], s["vz"]
      g1_v, g2_v = s["g1"], s["g2"]
      out_v = s["out"]

      hi_mask = jnp.full((L,), -65536, jnp.int32)

      @plsc.parallel_loop(0, groups, 1, unroll=2)
      def compute(i):
        base = i * L
        d = d_v[pl.ds(base, L)]
        vx = vx_v[pl.ds(base, L)]
        vy = vy_v[pl.ds(base, L)]
        vz = vz_v[pl.ds(base, L)]
        w1u = g1_v[pl.ds(base, L)]
        w2u = g2_v[pl.ds(base, L)]
        qu = lax.bitcast_convert_type(w1u & hi_mask, jnp.float32)
        mux = lax.bitcast_convert_type(lax.shift_left(w1u, 16), jnp.float32)
        muy = lax.bitcast_convert_type(w2u & hi_mask, jnp.float32)
        muz = lax.bitcast_convert_type(lax.shift_left(w2u, 16), jnp.float32)
        w1v = g1_v[pl.ds(chunk + base, L)]
        w2v = g2_v[pl.ds(chunk + base, L)]
        qv = lax.bitcast_convert_type(w1v & hi_mask, jnp.float32)
        mvx = lax.bitcast_convert_type(lax.shift_left(w1v, 16), jnp.float32)
        mvy = lax.bitcast_convert_type(w2v & hi_mask, jnp.float32)
        mvz = lax.bitcast_convert_type(lax.shift_left(w2v, 16), jnp.float32)

        x = jnp.clip(d * (1.0 / CUTOFF_SR), 0.0, 1.0)
        x2 = x * x
        x3 = x2 * x
        sw = 1.0 - (6.0 * x2 - 15.0 * x + 10.0) * x3
        inv_d = 1.0 / d
        chi = sw * _rsqrt(d * d + 1.0) + (1.0 - sw) * inv_d
        chi2 = chi * chi
        chi3 = chi2 * chi

        s1 = 1.0 / CUTOFF
        s2 = s1 * s1
        s3 = s2 * s1

        dot_uv = (vx * mvx + vy * mvy + vz * mvz) * inv_d
        dot_vu = (vx * mux + vy * muy + vz * muz) * inv_d
        mumu = mux * mvx + muy * mvy + muz * mvz

        e = qu * qv * (chi - s1)
        e = e + 2.0 * qu * dot_uv * (chi2 - s2)
        e = e + (mumu - 3.0 * dot_uv * dot_vu) * (chi3 - s3)
        e = KEHALF * e
        e = jnp.where(d <= CUTOFF, e, jnp.zeros_like(e))
        out_v[pl.ds(base, L)] = e

    fire_lin(sets[0], 0)
    drain_idx(sets[0])
    fire_gath(sets[0])
    fire_lin(sets[1], 1)

    def pair(jj, _):
      for p in (0, 1):
        s = sets[p]
        t = sets[1 - p]
        j = 2 * jj + p

        @pl.when(j + 1 < n_chunks)
        def _():
          drain_idx(t)
          fire_gath(t)

        drain_in(s)
        drain_gath(s)

        @pl.when(j >= 2)
        def _():
          drain_out(s)

        comp(s)
        fire_out(s, j)

        @pl.when(j + 2 < n_chunks)
        def _():
          fire_lin(s, j + 2)
      return ()

    lax.fori_loop(0, n_chunks // 2, pair, (), unroll=False)
    drain_out(sets[0])
    drain_out(sets[1])

  vm_f = pltpu.VMEM((chunk,), jnp.float32)
  vm_f2 = pltpu.VMEM((2 * chunk,), jnp.float32)
  vm_i2 = pltpu.VMEM((2 * chunk,), jnp.int32)
  sem = pltpu.SemaphoreType.DMA
  one_set = ([vm_i2] + [vm_f] * 4 + [vm_i2, vm_i2] + [vm_f]
             + [sem] * 4)

  return pl.kernel(
      body,
      out_type=jax.ShapeDtypeStruct((n_edges,), jnp.float32),
      mesh=mesh,
      scratch_types=(
          [pltpu.VMEM_SHARED((n_nodes,), jnp.int32),
           pltpu.VMEM_SHARED((n_nodes,), jnp.int32)]
          + one_set + one_set
      ),
  )


def kernel(distances_uv, atomic_charges, idx_u, idx_v, vectors_uv,
           atomic_dipoles):
  n_edges = distances_uv.shape[0]
  n_nodes = atomic_charges.shape[0]

  n_pad = (-n_nodes) % (NS * PIECE)
  q = atomic_charges
  dip = atomic_dipoles
  if n_pad:
    q = jnp.pad(q, (0, n_pad))
    dip = jnp.pad(dip, ((0, n_pad), (0, 0)))
  def b16(a):
    return lax.bitcast_convert_type(a.astype(jnp.bfloat16),
                                    jnp.uint16).astype(jnp.uint32)

  qx = ((b16(q) << 16) | b16(dip[:, 0])).astype(jnp.int32)
  yz = ((b16(dip[:, 1]) << 16) | b16(dip[:, 2])).astype(jnp.int32)

  iu = idx_u.astype(jnp.int32)
  iv = idx_v.astype(jnp.int32)
  d = distances_uv
  vec = vectors_uv

  e_pad = (-n_edges) % (NW * CHUNK * 2)
  if e_pad:
    d = jnp.pad(d, (0, e_pad), constant_values=1.0)
    vec = jnp.pad(vec, ((0, e_pad), (0, 0)))
    iu = jnp.pad(iu, (0, e_pad))
    iv = jnp.pad(iv, (0, e_pad))

  vec_t = vec.T
  vx = vec_t[0]
  vy = vec_t[1]
  vz = vec_t[2]

  fn = _build(q.shape[0], d.shape[0], CHUNK)
  out = fn(d, vx, vy, vz, iu, iv, qx, yz)
  return out[:n_edges] if e_pad else out
